# baseline (speedup 1.0000x reference)
import functools

import jax
import jax.numpy as jnp
from jax import lax
from jax.experimental import pallas as pl
from jax.experimental.pallas import tpu as pltpu
from jax.experimental.pallas import tpu_sc as plsc

B = 1024
S = 200
D = 64
V_PAPER = 1000000
N = B * S
NW = 32
CHUNK = 128
BLKS_PER_S = B // CHUNK
N_CHUNKS_TOT = N // CHUNK
PER_W = N_CHUNKS_TOT // NW
LANES = 16
GRP = CHUNK // LANES


def _sc_body(seq_hbm, pos_hbm, pap_hbm, tok_tab, pos_tab, papT_lin,
             w_hbm, b_hbm, out_hbm,
             idx_tok, idx_pos, idx_pap, buf_tok, buf_pos, pap_buf,
             pidx, out_buf, w_v, b_v, gsem, osem):
    wid = lax.axis_index("s") * 2 + lax.axis_index("c")

    pltpu.sync_copy(seq_hbm.at[wid], idx_tok)
    pltpu.sync_copy(pos_hbm.at[wid], idx_pos)
    pltpu.sync_copy(pap_hbm.at[wid], idx_pap)
    pltpu.sync_copy(w_hbm, w_v)
    pltpu.sync_copy(b_hbm, b_v)

    iota16 = lax.broadcasted_iota(jnp.int32, (LANES,), 0)
    rows_g = [iota16 + g * LANES for g in range(GRP)]

    def build_pidx(c, slot):
        segs = [idx_pap[c, pl.ds(g * LANES, LANES)] for g in range(GRP)]

        def jd_body(jd, carry):
            off = jd * V_PAPER
            for g in range(GRP):
                pidx[slot, jd, pl.ds(g * LANES, LANES)] = segs[g] + off
            return carry

        lax.fori_loop(0, D, jd_body, 0)

    def fire(c, slot):
        pltpu.async_copy(tok_tab.at[idx_tok.at[c]], buf_tok.at[slot], gsem)
        pltpu.async_copy(pos_tab.at[idx_pos.at[c]], buf_pos.at[slot], gsem)

        def row_fire(jd, carry):
            pltpu.async_copy(papT_lin.at[pidx.at[slot, jd]],
                             pap_buf.at[slot, jd], gsem)
            return carry

        lax.fori_loop(0, D, row_fire, 0)

    def drain(slot):
        pltpu.make_async_copy(tok_tab.at[idx_tok.at[0]], buf_tok.at[slot],
                              gsem).wait()
        pltpu.make_async_copy(pos_tab.at[idx_pos.at[0]], buf_pos.at[slot],
                              gsem).wait()
        pltpu.make_async_copy(out_hbm.at[0, :, pl.ds(0, CHUNK)],
                              pap_buf.at[slot], gsem).wait()

    def out_dst(c):
        chunk_id = wid * PER_W + c
        s = chunk_id // BLKS_PER_S
        b0 = (chunk_id % BLKS_PER_S) * CHUNK
        return out_hbm.at[s, :, pl.ds(b0, CHUNK)]

    def compute(c, slot):
        splat = jnp.full((LANES,), 0, jnp.int32)
        del splat

        def jd_body(jd, carry):
            jd_splat = jnp.full((LANES,), jd, jnp.int32)
            w0 = w_v[0, jd, :]
            w1 = w_v[1, jd, :]
            w2 = w_v[2, jd, :]
            bs = b_v[jd, :]
            for g in range(GRP):
                ds = pl.ds(g * LANES, LANES)
                tok_c = plsc.load_gather(buf_tok.at[slot], [rows_g[g], jd_splat])
                pos_c = plsc.load_gather(buf_pos.at[slot], [rows_g[g], jd_splat])
                pap_c = pap_buf[slot, jd, ds]
                acc = tok_c * w0 + pap_c * w1 + pos_c * w2 + bs
                out_buf[slot, jd, ds] = acc
            return carry

        lax.fori_loop(0, D, jd_body, 0)

    build_pidx(0, 0)
    fire(0, 0)

    def pair_body(i, carry):
        c0 = 2 * i
        c1 = c0 + 1

        drain(0)
        build_pidx(c1, 1)
        fire(c1, 1)
        compute(c0, 0)

        @pl.when(i > 0)
        def _():
            pltpu.make_async_copy(out_buf.at[0], out_dst(c0 - 2), osem).wait()

        pltpu.async_copy(out_buf.at[0], out_dst(c0), osem)

        drain(1)

        @pl.when(c1 + 1 < PER_W)
        def _():
            build_pidx(c1 + 1, 0)
            fire(c1 + 1, 0)

        compute(c1, 1)

        @pl.when(i > 0)
        def _():
            pltpu.make_async_copy(out_buf.at[1], out_dst(c1 - 2), osem).wait()

        pltpu.async_copy(out_buf.at[1], out_dst(c1), osem)
        return carry

    lax.fori_loop(0, PER_W // 2, pair_body, 0)

    pltpu.make_async_copy(out_buf.at[0], out_dst(PER_W - 2), osem).wait()
    pltpu.make_async_copy(out_buf.at[1], out_dst(PER_W - 1), osem).wait()


def kernel(sequence, position_ids, paper_ids, token_table, position_table,
           paper_table, embedding_weights, embedding_bias):
    seqT = sequence.T.reshape(NW, PER_W, CHUNK).astype(jnp.int32)
    posT = position_ids.T.reshape(NW, PER_W, CHUNK).astype(jnp.int32)
    papT = paper_ids.T.reshape(NW, PER_W, CHUNK).astype(jnp.int32)
    papT_lin = paper_table.T.reshape(D * V_PAPER)
    w_bc = jnp.broadcast_to(embedding_weights[:, :, None], (3, D, LANES))
    b_bc = jnp.broadcast_to(embedding_bias[:, None], (D, LANES))

    mesh = plsc.VectorSubcoreMesh(core_axis_name="c", subcore_axis_name="s")
    run = functools.partial(
        pl.kernel,
        mesh=mesh,
        compiler_params=pltpu.CompilerParams(use_tc_tiling_on_sc=False,
                                             needs_layout_passes=False),
        out_type=jax.ShapeDtypeStruct((S, D, B), jnp.float32),
        scratch_types=[
            pltpu.VMEM((PER_W, CHUNK), jnp.int32),
            pltpu.VMEM((PER_W, CHUNK), jnp.int32),
            pltpu.VMEM((PER_W, CHUNK), jnp.int32),
            pltpu.VMEM((2, CHUNK, D), jnp.float32),
            pltpu.VMEM((2, CHUNK, D), jnp.float32),
            pltpu.VMEM((2, D, CHUNK), jnp.float32),
            pltpu.VMEM((2, D, CHUNK), jnp.int32),
            pltpu.VMEM((2, D, CHUNK), jnp.float32),
            pltpu.VMEM((3, D, LANES), jnp.float32),
            pltpu.VMEM((D, LANES), jnp.float32),
            pltpu.SemaphoreType.DMA,
            pltpu.SemaphoreType.DMA,
        ],
    )(_sc_body)
    out = run(seqT, posT, papT, token_table, position_table, papT_lin,
              w_bc, b_bc)
    return jnp.transpose(out, (2, 0, 1))


# speedup vs baseline: 1.0407x; 1.0407x over previous
import functools

import jax
import jax.numpy as jnp
from jax import lax
from jax.experimental import pallas as pl
from jax.experimental.pallas import tpu as pltpu
from jax.experimental.pallas import tpu_sc as plsc

B = 1024
S = 200
D = 64
V_PAPER = 1000000
N = B * S
NW = 32
CHUNK = 128
BLKS_PER_S = B // CHUNK
N_CHUNKS_TOT = N // CHUNK
PER_W = N_CHUNKS_TOT // NW
LANES = 16
GRP = CHUNK // LANES


def _sc_body(seq_hbm, pos_hbm, pap_hbm, tok_tab, pos_tab, papT_lin,
             w_hbm, b_hbm, out_hbm,
             idx_tok, idx_pos, idx_pap, buf_tok, buf_pos, pap_buf,
             pidx, out_buf, w_v, b_v, gsem, osem):
    wid = lax.axis_index("s") * 2 + lax.axis_index("c")

    pltpu.sync_copy(seq_hbm.at[wid], idx_tok)
    pltpu.sync_copy(pos_hbm.at[wid], idx_pos)
    pltpu.sync_copy(pap_hbm.at[wid], idx_pap)
    pltpu.sync_copy(w_hbm, w_v)
    pltpu.sync_copy(b_hbm, b_v)

    iota16 = lax.broadcasted_iota(jnp.int32, (LANES,), 0)
    rows_g = [iota16 + g * LANES for g in range(GRP)]

    def build_pidx(c, slot):
        segs = [idx_pap[c, pl.ds(g * LANES, LANES)] for g in range(GRP)]

        def jd_body(jd, carry):
            off = jd * V_PAPER
            base = jd * CHUNK
            for g in range(GRP):
                pidx[slot, pl.ds(base + g * LANES, LANES)] = segs[g] + off
            return carry

        lax.fori_loop(0, D, jd_body, 0)

    def fire(c, slot):
        pltpu.async_copy(tok_tab.at[idx_tok.at[c]], buf_tok.at[slot], gsem)
        pltpu.async_copy(pos_tab.at[idx_pos.at[c]], buf_pos.at[slot], gsem)

        pltpu.async_copy(papT_lin.at[pidx.at[slot]], pap_buf.at[slot], gsem)

    def drain(slot):
        pltpu.make_async_copy(tok_tab.at[idx_tok.at[0]], buf_tok.at[slot],
                              gsem).wait()
        pltpu.make_async_copy(pos_tab.at[idx_pos.at[0]], buf_pos.at[slot],
                              gsem).wait()
        pltpu.make_async_copy(papT_lin.at[pidx.at[slot]], pap_buf.at[slot],
                              gsem).wait()

    def out_dst(c):
        chunk_id = wid * PER_W + c
        s = chunk_id // BLKS_PER_S
        b0 = (chunk_id % BLKS_PER_S) * CHUNK
        return out_hbm.at[s, :, pl.ds(b0, CHUNK)]

    def compute(c, slot):
        splat = jnp.full((LANES,), 0, jnp.int32)
        del splat

        def jd_body(jd, carry):
            jd_splat = jnp.full((LANES,), jd, jnp.int32)
            w0 = w_v[0, jd, :]
            w1 = w_v[1, jd, :]
            w2 = w_v[2, jd, :]
            bs = b_v[jd, :]
            for g in range(GRP):
                ds = pl.ds(g * LANES, LANES)
                tok_c = plsc.load_gather(buf_tok.at[slot], [rows_g[g], jd_splat])
                pos_c = plsc.load_gather(buf_pos.at[slot], [rows_g[g], jd_splat])
                pap_c = pap_buf[slot, pl.ds(jd * CHUNK + g * LANES, LANES)]
                acc = tok_c * w0 + pap_c * w1 + pos_c * w2 + bs
                out_buf[slot, jd, ds] = acc
            return carry

        lax.fori_loop(0, D, jd_body, 0)

    build_pidx(0, 0)
    fire(0, 0)

    def pair_body(i, carry):
        c0 = 2 * i
        c1 = c0 + 1

        drain(0)
        build_pidx(c1, 1)
        fire(c1, 1)
        compute(c0, 0)

        @pl.when(i > 0)
        def _():
            pltpu.make_async_copy(out_buf.at[0], out_dst(c0 - 2), osem).wait()

        pltpu.async_copy(out_buf.at[0], out_dst(c0), osem)

        drain(1)

        @pl.when(c1 + 1 < PER_W)
        def _():
            build_pidx(c1 + 1, 0)
            fire(c1 + 1, 0)

        compute(c1, 1)

        @pl.when(i > 0)
        def _():
            pltpu.make_async_copy(out_buf.at[1], out_dst(c1 - 2), osem).wait()

        pltpu.async_copy(out_buf.at[1], out_dst(c1), osem)
        return carry

    lax.fori_loop(0, PER_W // 2, pair_body, 0)

    pltpu.make_async_copy(out_buf.at[0], out_dst(PER_W - 2), osem).wait()
    pltpu.make_async_copy(out_buf.at[1], out_dst(PER_W - 1), osem).wait()


def kernel(sequence, position_ids, paper_ids, token_table, position_table,
           paper_table, embedding_weights, embedding_bias):
    seqT = sequence.T.reshape(NW, PER_W, CHUNK).astype(jnp.int32)
    posT = position_ids.T.reshape(NW, PER_W, CHUNK).astype(jnp.int32)
    papT = paper_ids.T.reshape(NW, PER_W, CHUNK).astype(jnp.int32)
    papT_lin = paper_table.T.reshape(D * V_PAPER)
    w_bc = jnp.broadcast_to(embedding_weights[:, :, None], (3, D, LANES))
    b_bc = jnp.broadcast_to(embedding_bias[:, None], (D, LANES))

    mesh = plsc.VectorSubcoreMesh(core_axis_name="c", subcore_axis_name="s")
    run = functools.partial(
        pl.kernel,
        mesh=mesh,
        compiler_params=pltpu.CompilerParams(use_tc_tiling_on_sc=False,
                                             needs_layout_passes=False),
        out_type=jax.ShapeDtypeStruct((S, D, B), jnp.float32),
        scratch_types=[
            pltpu.VMEM((PER_W, CHUNK), jnp.int32),
            pltpu.VMEM((PER_W, CHUNK), jnp.int32),
            pltpu.VMEM((PER_W, CHUNK), jnp.int32),
            pltpu.VMEM((2, CHUNK, D), jnp.float32),
            pltpu.VMEM((2, CHUNK, D), jnp.float32),
            pltpu.VMEM((2, D * CHUNK), jnp.float32),
            pltpu.VMEM((2, D * CHUNK), jnp.int32),
            pltpu.VMEM((2, D, CHUNK), jnp.float32),
            pltpu.VMEM((3, D, LANES), jnp.float32),
            pltpu.VMEM((D, LANES), jnp.float32),
            pltpu.SemaphoreType.DMA,
            pltpu.SemaphoreType.DMA,
        ],
    )(_sc_body)
    out = run(seqT, posT, papT, token_table, position_table, papT_lin,
              w_bc, b_bc)
    return jnp.transpose(out, (2, 0, 1))


# speedup vs baseline: 3.8123x; 3.6632x over previous
import functools

import jax
import jax.numpy as jnp
from jax import lax
from jax.experimental import pallas as pl
from jax.experimental.pallas import tpu as pltpu
from jax.experimental.pallas import tpu_sc as plsc

B = 1024
S = 200
D = 64
N = B * S
NW = 32
CHUNK = 128
BLKS_PER_S = B // CHUNK
PER_W = (N // CHUNK) // NW
LANES = 16
GRP = CHUNK // LANES


def _sc_body(seq_hbm, pos_hbm, pap_hbm, tok_tab, pos_tab, pap_tab,
             w_hbm, b_hbm, out_hbm,
             idx_tok, idx_pos, idx_pap, buf_tok, buf_pap, pos_local,
             out_buf, w_v, b_v, gsem, osem):
    wid = lax.axis_index("s") * 2 + lax.axis_index("c")

    pltpu.sync_copy(seq_hbm.at[wid], idx_tok)
    pltpu.sync_copy(pos_hbm.at[wid], idx_pos)
    pltpu.sync_copy(pap_hbm.at[wid], idx_pap)
    pltpu.sync_copy(w_hbm, w_v)
    pltpu.sync_copy(b_hbm, b_v)
    pltpu.sync_copy(pos_tab, pos_local)

    iota16 = lax.broadcasted_iota(jnp.int32, (LANES,), 0)
    rows_g = [iota16 + g * LANES for g in range(GRP)]

    def fire(c, slot):
        pltpu.async_copy(tok_tab.at[idx_tok.at[c]], buf_tok.at[slot], gsem)
        pltpu.async_copy(pap_tab.at[idx_pap.at[c]], buf_pap.at[slot], gsem)

    def drain(slot):
        pltpu.make_async_copy(tok_tab.at[idx_tok.at[0]], buf_tok.at[slot],
                              gsem).wait()
        pltpu.make_async_copy(pap_tab.at[idx_pap.at[0]], buf_pap.at[slot],
                              gsem).wait()

    def out_dst(c):
        chunk_id = wid * PER_W + c
        s = chunk_id // BLKS_PER_S
        b0 = (chunk_id % BLKS_PER_S) * CHUNK
        return out_hbm.at[s, :, pl.ds(b0, CHUNK)]

    def compute(c, slot):
        pos_idx = [idx_pos[c, pl.ds(g * LANES, LANES)] for g in range(GRP)]

        def jd_body(jd, carry):
            jd_splat = jnp.full((LANES,), jd, jnp.int32)
            w0 = w_v[0, jd, :]
            w1 = w_v[1, jd, :]
            w2 = w_v[2, jd, :]
            bs = b_v[jd, :]
            for g in range(GRP):
                tok_c = plsc.load_gather(buf_tok.at[slot],
                                         [rows_g[g], jd_splat])
                pap_c = plsc.load_gather(buf_pap.at[slot],
                                         [rows_g[g], jd_splat])
                pos_c = plsc.load_gather(pos_local, [pos_idx[g], jd_splat])
                acc = tok_c * w0 + pap_c * w1 + pos_c * w2 + bs
                out_buf[slot, jd, pl.ds(g * LANES, LANES)] = acc
            return carry

        lax.fori_loop(0, D, jd_body, 0)

    fire(0, 0)

    def pair_body(i, carry):
        c0 = 2 * i
        c1 = c0 + 1

        drain(0)
        fire(c1, 1)
        compute(c0, 0)

        @pl.when(i > 0)
        def _():
            pltpu.make_async_copy(out_buf.at[0], out_dst(c0 - 2), osem).wait()

        pltpu.async_copy(out_buf.at[0], out_dst(c0), osem)

        drain(1)

        @pl.when(c1 + 1 < PER_W)
        def _():
            fire(c1 + 1, 0)

        compute(c1, 1)

        @pl.when(i > 0)
        def _():
            pltpu.make_async_copy(out_buf.at[1], out_dst(c1 - 2), osem).wait()

        pltpu.async_copy(out_buf.at[1], out_dst(c1), osem)
        return carry

    lax.fori_loop(0, PER_W // 2, pair_body, 0)

    pltpu.make_async_copy(out_buf.at[0], out_dst(PER_W - 2), osem).wait()
    pltpu.make_async_copy(out_buf.at[1], out_dst(PER_W - 1), osem).wait()


def kernel(sequence, position_ids, paper_ids, token_table, position_table,
           paper_table, embedding_weights, embedding_bias):
    seqT = sequence.T.reshape(NW, PER_W, CHUNK).astype(jnp.int32)
    posT = position_ids.T.reshape(NW, PER_W, CHUNK).astype(jnp.int32)
    papT = paper_ids.T.reshape(NW, PER_W, CHUNK).astype(jnp.int32)
    w_bc = jnp.broadcast_to(embedding_weights[:, :, None], (3, D, LANES))
    b_bc = jnp.broadcast_to(embedding_bias[:, None], (D, LANES))

    mesh = plsc.VectorSubcoreMesh(core_axis_name="c", subcore_axis_name="s")
    run = functools.partial(
        pl.kernel,
        mesh=mesh,
        compiler_params=pltpu.CompilerParams(use_tc_tiling_on_sc=False,
                                             needs_layout_passes=False),
        out_type=jax.ShapeDtypeStruct((S, D, B), jnp.float32),
        scratch_types=[
            pltpu.VMEM((PER_W, CHUNK), jnp.int32),
            pltpu.VMEM((PER_W, CHUNK), jnp.int32),
            pltpu.VMEM((PER_W, CHUNK), jnp.int32),
            pltpu.VMEM((2, CHUNK, D), jnp.float32),
            pltpu.VMEM((2, CHUNK, D), jnp.float32),
            pltpu.VMEM((S, D), jnp.float32),
            pltpu.VMEM((2, D, CHUNK), jnp.float32),
            pltpu.VMEM((3, D, LANES), jnp.float32),
            pltpu.VMEM((D, LANES), jnp.float32),
            pltpu.SemaphoreType.DMA,
            pltpu.SemaphoreType.DMA,
        ],
    )(_sc_body)
    out = run(seqT, posT, papT, token_table, position_table, paper_table,
              w_bc, b_bc)
    return jnp.transpose(out, (2, 0, 1))


# speedup vs baseline: 3.8550x; 1.0112x over previous
import functools

import jax
import jax.numpy as jnp
from jax import lax
from jax.experimental import pallas as pl
from jax.experimental.pallas import tpu as pltpu
from jax.experimental.pallas import tpu_sc as plsc

B = 1024
S = 200
D = 64
N = B * S
NW = 32
CHUNK = 128
BLKS_PER_S = B // CHUNK
PER_W = (N // CHUNK) // NW
LANES = 16
GRP = CHUNK // LANES


def _sc_body(seq_hbm, pos_hbm, pap_hbm, tok_tab, pos_tab, pap_tab,
             w_hbm, b_hbm, out_hbm,
             idx_tok, idx_pos, idx_pap, buf_tok, buf_pap, pos_local,
             out_buf, w_v, b_v, gsem, osem):
    wid = lax.axis_index("s") * 2 + lax.axis_index("c")

    pltpu.sync_copy(seq_hbm.at[wid], idx_tok)
    pltpu.sync_copy(pos_hbm.at[wid], idx_pos)
    pltpu.sync_copy(pap_hbm.at[wid], idx_pap)
    pltpu.sync_copy(w_hbm, w_v)
    pltpu.sync_copy(b_hbm, b_v)
    pltpu.sync_copy(pos_tab, pos_local)

    iota16 = lax.broadcasted_iota(jnp.int32, (LANES,), 0)
    rows_g = [iota16 + g * LANES for g in range(GRP)]

    def fire(c, slot):
        pltpu.async_copy(tok_tab.at[idx_tok.at[c]], buf_tok.at[slot], gsem)
        pltpu.async_copy(pap_tab.at[idx_pap.at[c]], buf_pap.at[slot], gsem)

    def drain(slot):
        pltpu.make_async_copy(tok_tab.at[idx_tok.at[0]], buf_tok.at[slot],
                              gsem).wait()
        pltpu.make_async_copy(pap_tab.at[idx_pap.at[0]], buf_pap.at[slot],
                              gsem).wait()

    def out_dst(c):
        return out_hbm.at[wid * PER_W + c]

    def compute(c, slot):
        pos_idx = [idx_pos[c, pl.ds(g * LANES, LANES)] for g in range(GRP)]

        def jd_body(jd, carry):
            jd_splat = jnp.full((LANES,), jd, jnp.int32)
            w0 = w_v[0, jd, :]
            w1 = w_v[1, jd, :]
            w2 = w_v[2, jd, :]
            bs = b_v[jd, :]
            for g in range(GRP):
                tok_c = plsc.load_gather(buf_tok.at[slot],
                                         [rows_g[g], jd_splat])
                pap_c = plsc.load_gather(buf_pap.at[slot],
                                         [rows_g[g], jd_splat])
                pos_c = plsc.load_gather(pos_local, [pos_idx[g], jd_splat])
                acc = tok_c * w0 + pap_c * w1 + pos_c * w2 + bs
                out_buf[slot, jd, pl.ds(g * LANES, LANES)] = acc
            return carry

        lax.fori_loop(0, D, jd_body, 0)

    fire(0, 0)

    def pair_body(i, carry):
        c0 = 2 * i
        c1 = c0 + 1

        drain(0)
        fire(c1, 1)
        compute(c0, 0)

        @pl.when(i > 0)
        def _():
            pltpu.make_async_copy(out_buf.at[0], out_dst(c0 - 2), osem).wait()

        pltpu.async_copy(out_buf.at[0], out_dst(c0), osem)

        drain(1)

        @pl.when(c1 + 1 < PER_W)
        def _():
            fire(c1 + 1, 0)

        compute(c1, 1)

        @pl.when(i > 0)
        def _():
            pltpu.make_async_copy(out_buf.at[1], out_dst(c1 - 2), osem).wait()

        pltpu.async_copy(out_buf.at[1], out_dst(c1), osem)
        return carry

    lax.fori_loop(0, PER_W // 2, pair_body, 0)

    pltpu.make_async_copy(out_buf.at[0], out_dst(PER_W - 2), osem).wait()
    pltpu.make_async_copy(out_buf.at[1], out_dst(PER_W - 1), osem).wait()


def kernel(sequence, position_ids, paper_ids, token_table, position_table,
           paper_table, embedding_weights, embedding_bias):
    seqT = sequence.T.reshape(NW, PER_W, CHUNK).astype(jnp.int32)
    posT = position_ids.T.reshape(NW, PER_W, CHUNK).astype(jnp.int32)
    papT = paper_ids.T.reshape(NW, PER_W, CHUNK).astype(jnp.int32)
    w_bc = jnp.broadcast_to(embedding_weights[:, :, None], (3, D, LANES))
    b_bc = jnp.broadcast_to(embedding_bias[:, None], (D, LANES))

    mesh = plsc.VectorSubcoreMesh(core_axis_name="c", subcore_axis_name="s")
    run = functools.partial(
        pl.kernel,
        mesh=mesh,
        compiler_params=pltpu.CompilerParams(use_tc_tiling_on_sc=False,
                                             needs_layout_passes=False),
        out_type=jax.ShapeDtypeStruct((N // CHUNK, D, CHUNK), jnp.float32),
        scratch_types=[
            pltpu.VMEM((PER_W, CHUNK), jnp.int32),
            pltpu.VMEM((PER_W, CHUNK), jnp.int32),
            pltpu.VMEM((PER_W, CHUNK), jnp.int32),
            pltpu.VMEM((2, CHUNK, D), jnp.float32),
            pltpu.VMEM((2, CHUNK, D), jnp.float32),
            pltpu.VMEM((S, D), jnp.float32),
            pltpu.VMEM((2, D, CHUNK), jnp.float32),
            pltpu.VMEM((3, D, LANES), jnp.float32),
            pltpu.VMEM((D, LANES), jnp.float32),
            pltpu.SemaphoreType.DMA,
            pltpu.SemaphoreType.DMA,
        ],
    )(_sc_body)
    out = run(seqT, posT, papT, token_table, position_table, paper_table,
              w_bc, b_bc)
    out = out.reshape(S, BLKS_PER_S, D, CHUNK)
    return jnp.transpose(out, (1, 3, 0, 2)).reshape(B, S, D)


# speedup vs baseline: 5.7858x; 1.5009x over previous
import functools

import jax
import jax.numpy as jnp
from jax import lax
from jax.experimental import pallas as pl
from jax.experimental.pallas import tpu as pltpu
from jax.experimental.pallas import tpu_sc as plsc

B = 1024
S = 200
D = 64
N = B * S
NW = 32
PER_W = N // NW
CHUNK = 128
N_CHUNKS = PER_W // CHUNK
LANES = 16
DBLK = D // LANES
NBUF = 2


def _sc_body(seq_hbm, pos_hbm, pap_hbm, tok_tab, pos_tab, pap_tab,
             w_hbm, b_hbm, out_hbm,
             idx_tok, idx_pos, idx_pap, buf_tok, pos_local, buf_pap,
             buf_out, w_v, b_v, gsem, osem):
    wid = lax.axis_index("s") * 2 + lax.axis_index("c")

    pltpu.sync_copy(seq_hbm.at[wid], idx_tok)
    pltpu.sync_copy(pos_hbm.at[wid], idx_pos)
    pltpu.sync_copy(pap_hbm.at[wid], idx_pap)
    pltpu.sync_copy(pos_tab, pos_local)
    pltpu.sync_copy(w_hbm, w_v)
    pltpu.sync_copy(b_hbm, b_v)

    iota16 = lax.broadcasted_iota(jnp.int32, (LANES,), 0)
    cols_j = [iota16 + j * LANES for j in range(DBLK)]

    w_tok = [w_v[0, pl.ds(j * LANES, LANES)] for j in range(DBLK)]
    w_pap = [w_v[1, pl.ds(j * LANES, LANES)] for j in range(DBLK)]
    w_pos = [w_v[2, pl.ds(j * LANES, LANES)] for j in range(DBLK)]
    bias = [b_v[pl.ds(j * LANES, LANES)] for j in range(DBLK)]

    def fire(c, slot):
        pltpu.async_copy(tok_tab.at[idx_tok.at[c]], buf_tok.at[slot], gsem)
        pltpu.async_copy(pap_tab.at[idx_pap.at[c]], buf_pap.at[slot], gsem)

    def drain_gathers(slot):
        pltpu.make_async_copy(tok_tab.at[idx_tok.at[0]], buf_tok.at[slot],
                              gsem).wait()
        pltpu.make_async_copy(pap_tab.at[idx_pap.at[0]], buf_pap.at[slot],
                              gsem).wait()

    fire(0, 0)

    def chunk_body(c, carry):
        slot = c % NBUF
        drain_gathers(slot)

        @pl.when(c + 1 < N_CHUNKS)
        def _():
            fire(c + 1, (c + 1) % NBUF)

        tok, pap, out = (buf_tok.at[slot], buf_pap.at[slot], buf_out.at[slot])
        c_splat = jnp.full((LANES,), c, jnp.int32)

        def row_body(r, carry2):
            r_splat = jnp.full((LANES,), r, jnp.int32)
            p_splat = plsc.load_gather(idx_pos, [c_splat, r_splat])
            for j in range(DBLK):
                ds = pl.ds(j * LANES, LANES)
                acc = tok[r, ds] * w_tok[j]
                acc += pap[r, ds] * w_pap[j]
                acc += plsc.load_gather(pos_local, [p_splat, cols_j[j]]) \
                    * w_pos[j]
                out[r, ds] = acc + bias[j]
            return carry2

        lax.fori_loop(0, CHUNK, row_body, 0, unroll=2)

        dst = out_hbm.at[pl.ds(wid * PER_W + c * CHUNK, CHUNK), :]

        @pl.when(c >= NBUF)
        def _():
            pltpu.make_async_copy(out, dst, osem).wait()

        pltpu.async_copy(out, dst, osem)
        return carry

    lax.fori_loop(0, N_CHUNKS, chunk_body, 0)

    for t in range(NBUF):
        c = N_CHUNKS - NBUF + t
        pltpu.make_async_copy(
            buf_out.at[c % NBUF],
            out_hbm.at[pl.ds(wid * PER_W + c * CHUNK, CHUNK), :],
            osem).wait()


def kernel(sequence, position_ids, paper_ids, token_table, position_table,
           paper_table, embedding_weights, embedding_bias):
    seq3d = sequence.reshape(NW, N_CHUNKS, CHUNK).astype(jnp.int32)
    pos3d = position_ids.reshape(NW, N_CHUNKS, CHUNK).astype(jnp.int32)
    pap3d = paper_ids.reshape(NW, N_CHUNKS, CHUNK).astype(jnp.int32)

    mesh = plsc.VectorSubcoreMesh(core_axis_name="c", subcore_axis_name="s")
    run = functools.partial(
        pl.kernel,
        mesh=mesh,
        compiler_params=pltpu.CompilerParams(use_tc_tiling_on_sc=False,
                                             needs_layout_passes=False),
        out_type=jax.ShapeDtypeStruct((N, D), jnp.float32),
        scratch_types=[
            pltpu.VMEM((N_CHUNKS, CHUNK), jnp.int32),
            pltpu.VMEM((N_CHUNKS, CHUNK), jnp.int32),
            pltpu.VMEM((N_CHUNKS, CHUNK), jnp.int32),
            pltpu.VMEM((NBUF, CHUNK, D), jnp.float32),
            pltpu.VMEM((S, D), jnp.float32),
            pltpu.VMEM((NBUF, CHUNK, D), jnp.float32),
            pltpu.VMEM((NBUF, CHUNK, D), jnp.float32),
            pltpu.VMEM((3, D), jnp.float32),
            pltpu.VMEM((D,), jnp.float32),
            pltpu.SemaphoreType.DMA,
            pltpu.SemaphoreType.DMA,
        ],
    )(_sc_body)
    out = run(seq3d, pos3d, pap3d, token_table, position_table, paper_table,
              embedding_weights, embedding_bias)
    return out.reshape(B, S, D)


# speedup vs baseline: 6.0162x; 1.0398x over previous
import functools

import jax
import jax.numpy as jnp
from jax import lax
from jax.experimental import pallas as pl
from jax.experimental.pallas import tpu as pltpu
from jax.experimental.pallas import tpu_sc as plsc

B = 1024
S = 200
D = 64
N = B * S
NW = 32
PER_W = N // NW
CHUNK = 128
N_CHUNKS = PER_W // CHUNK
LANES = 16
DBLK = D // LANES
NBUF = 2


def _sc_body(seq_hbm, pos_hbm, pap_hbm, tok_tab, pos_tab, pap_tab,
             w_hbm, b_hbm, out_hbm,
             idx_tok, idx_pos, idx_pap, buf_tok, buf_pos, buf_pap,
             buf_out, w_v, b_v, gsem, osem):
    wid = lax.axis_index("s") * 2 + lax.axis_index("c")

    pltpu.sync_copy(seq_hbm.at[wid], idx_tok)
    pltpu.sync_copy(pos_hbm.at[wid], idx_pos)
    pltpu.sync_copy(pap_hbm.at[wid], idx_pap)
    pltpu.sync_copy(w_hbm, w_v)
    pltpu.sync_copy(b_hbm, b_v)

    w_tok = [w_v[0, pl.ds(j * LANES, LANES)] for j in range(DBLK)]
    w_pap = [w_v[1, pl.ds(j * LANES, LANES)] for j in range(DBLK)]
    w_pos = [w_v[2, pl.ds(j * LANES, LANES)] for j in range(DBLK)]
    bias = [b_v[pl.ds(j * LANES, LANES)] for j in range(DBLK)]

    def fire(c, slot):
        pltpu.async_copy(tok_tab.at[idx_tok.at[c]], buf_tok.at[slot], gsem)
        pltpu.async_copy(pap_tab.at[idx_pap.at[c]], buf_pap.at[slot], gsem)
        pltpu.async_copy(pos_tab.at[idx_pos.at[c]], buf_pos.at[slot], gsem)

    def drain_gathers(slot):
        pltpu.make_async_copy(tok_tab.at[idx_tok.at[0]], buf_tok.at[slot],
                              gsem).wait()
        pltpu.make_async_copy(pap_tab.at[idx_pap.at[0]], buf_pap.at[slot],
                              gsem).wait()
        pltpu.make_async_copy(pos_tab.at[idx_pos.at[0]], buf_pos.at[slot],
                              gsem).wait()

    fire(0, 0)

    def chunk_body(c, carry):
        slot = c % NBUF
        drain_gathers(slot)

        @pl.when(c + 1 < N_CHUNKS)
        def _():
            fire(c + 1, (c + 1) % NBUF)

        tok, pap, pos, out = (buf_tok.at[slot], buf_pap.at[slot],
                              buf_pos.at[slot], buf_out.at[slot])

        def row_body(r, carry2):
            for j in range(DBLK):
                ds = pl.ds(j * LANES, LANES)
                acc = tok[r, ds] * w_tok[j]
                acc += pap[r, ds] * w_pap[j]
                acc += pos[r, ds] * w_pos[j]
                out[r, ds] = acc + bias[j]
            return carry2

        lax.fori_loop(0, CHUNK, row_body, 0, unroll=4)

        dst = out_hbm.at[pl.ds(wid * PER_W + c * CHUNK, CHUNK), :]

        @pl.when(c >= NBUF)
        def _():
            pltpu.make_async_copy(out, dst, osem).wait()

        pltpu.async_copy(out, dst, osem)
        return carry

    lax.fori_loop(0, N_CHUNKS, chunk_body, 0)

    for t in range(NBUF):
        c = N_CHUNKS - NBUF + t
        pltpu.make_async_copy(
            buf_out.at[c % NBUF],
            out_hbm.at[pl.ds(wid * PER_W + c * CHUNK, CHUNK), :],
            osem).wait()


def kernel(sequence, position_ids, paper_ids, token_table, position_table,
           paper_table, embedding_weights, embedding_bias):
    seq3d = sequence.reshape(NW, N_CHUNKS, CHUNK).astype(jnp.int32)
    pos3d = position_ids.reshape(NW, N_CHUNKS, CHUNK).astype(jnp.int32)
    pap3d = paper_ids.reshape(NW, N_CHUNKS, CHUNK).astype(jnp.int32)

    mesh = plsc.VectorSubcoreMesh(core_axis_name="c", subcore_axis_name="s")
    run = functools.partial(
        pl.kernel,
        mesh=mesh,
        compiler_params=pltpu.CompilerParams(use_tc_tiling_on_sc=False),
        out_type=jax.ShapeDtypeStruct((N, D), jnp.float32),
        scratch_types=[
            pltpu.VMEM((N_CHUNKS, CHUNK), jnp.int32),
            pltpu.VMEM((N_CHUNKS, CHUNK), jnp.int32),
            pltpu.VMEM((N_CHUNKS, CHUNK), jnp.int32),
            pltpu.VMEM((NBUF, CHUNK, D), jnp.float32),
            pltpu.VMEM((NBUF, CHUNK, D), jnp.float32),
            pltpu.VMEM((NBUF, CHUNK, D), jnp.float32),
            pltpu.VMEM((NBUF, CHUNK, D), jnp.float32),
            pltpu.VMEM((3, D), jnp.float32),
            pltpu.VMEM((D,), jnp.float32),
            pltpu.SemaphoreType.DMA,
            pltpu.SemaphoreType.DMA,
        ],
    )(_sc_body)
    out = run(seq3d, pos3d, pap3d, token_table, position_table, paper_table,
              embedding_weights, embedding_bias)
    return out.reshape(B, S, D)
